# baseline (speedup 1.0000x reference)
import jax
import jax.numpy as jnp
from jax import lax
from jax.experimental import pallas as pl
from jax.experimental.pallas import tpu as pltpu
from jax.experimental.pallas import tpu_sc as plsc

BATCH = 16384
LANES = 16

_NC = 2
_NS = 16
_NW = _NC * _NS
_CHUNK = BATCH // _NW


def _sc_gather_kernel(lg_hbm, sn_hbm, hts_hbm, ats_hbm,
                      gam_hbm, hfa_hbm, del_hbm, att_hbm, defn_hbm,
                      bh_hbm, ba_hbm,
                      lg_v, sn_v, hts_v, ats_v,
                      ah_v, aa_v, dh_v, da_v,
                      gam_t, hfa_t, del_t,
                      bh_v, ba_v, sem):
    wid = lax.axis_index("s") * _NC + lax.axis_index("c")
    base = wid * _CHUNK

    pltpu.sync_copy(lg_hbm.at[pl.ds(base, _CHUNK)], lg_v)
    pltpu.sync_copy(sn_hbm.at[pl.ds(base, _CHUNK)], sn_v)
    pltpu.sync_copy(hts_hbm.at[pl.ds(base, _CHUNK)], hts_v)
    pltpu.sync_copy(ats_hbm.at[pl.ds(base, _CHUNK)], ats_v)

    copies = [
        pltpu.async_copy(att_hbm.at[hts_v], ah_v, sem),
        pltpu.async_copy(att_hbm.at[ats_v], aa_v, sem),
        pltpu.async_copy(defn_hbm.at[hts_v], dh_v, sem),
        pltpu.async_copy(defn_hbm.at[ats_v], da_v, sem),
    ]

    pltpu.sync_copy(gam_hbm, gam_t.at[pl.ds(0, 1000)])
    pltpu.sync_copy(hfa_hbm, hfa_t.at[pl.ds(0, 1000)])
    pltpu.sync_copy(del_hbm, del_t.at[pl.ds(0, 50)])

    for c in copies:
        c.wait()

    for i in range(_CHUNK // LANES):
        s = i * LANES
        lv = lg_v[pl.ds(s, LANES)]
        sv = sn_v[pl.ds(s, LANES)]
        g = plsc.load_gather(gam_t, [lv])
        h = plsc.load_gather(hfa_t, [lv])
        d = plsc.load_gather(del_t, [sv])
        gd = g + d
        bh_v[pl.ds(s, LANES)] = gd + h + ah_v[pl.ds(s, LANES)] - da_v[pl.ds(s, LANES)]
        ba_v[pl.ds(s, LANES)] = gd + aa_v[pl.ds(s, LANES)] - dh_v[pl.ds(s, LANES)]

    pltpu.sync_copy(bh_v, bh_hbm.at[pl.ds(base, _CHUNK)])
    pltpu.sync_copy(ba_v, ba_hbm.at[pl.ds(base, _CHUNK)])


def _sc_gather(lg, sn, hts, ats, gam2, hfa2, del2, att, defn):
    mesh = plsc.VectorSubcoreMesh(core_axis_name="c", subcore_axis_name="s")
    f = pl.kernel(
        _sc_gather_kernel,
        mesh=mesh,
        compiler_params=pltpu.CompilerParams(needs_layout_passes=False),
        out_type=[
            jax.ShapeDtypeStruct((BATCH,), jnp.float32),
            jax.ShapeDtypeStruct((BATCH,), jnp.float32),
        ],
        scratch_types=[
            pltpu.VMEM((_CHUNK,), jnp.int32),
            pltpu.VMEM((_CHUNK,), jnp.int32),
            pltpu.VMEM((_CHUNK,), jnp.int32),
            pltpu.VMEM((_CHUNK,), jnp.int32),
            pltpu.VMEM((_CHUNK,), jnp.float32),
            pltpu.VMEM((_CHUNK,), jnp.float32),
            pltpu.VMEM((_CHUNK,), jnp.float32),
            pltpu.VMEM((_CHUNK,), jnp.float32),
            pltpu.VMEM((1024,), jnp.float32),
            pltpu.VMEM((1024,), jnp.float32),
            pltpu.VMEM((64,), jnp.float32),
            pltpu.VMEM((_CHUNK,), jnp.float32),
            pltpu.VMEM((_CHUNK,), jnp.float32),
            pltpu.SemaphoreType.DMA,
        ],
    )
    return f(lg, sn, hts, ats, gam2, hfa2, del2, att, defn)


def _tc_combine_kernel(xh_ref, xa_ref, beh_ref, bea_ref, mu_ref,
                       bh_ref, ba_ref, oh_ref, oa_ref):
    mu = mu_ref[0, 0]
    lin_h = jnp.sum(xh_ref[...] * beh_ref[...], axis=0)
    lin_a = jnp.sum(xa_ref[...] * bea_ref[...], axis=0)
    log_h = jnp.clip(bh_ref[0, 0, :] + mu + lin_h, -10.0, 10.0)
    log_a = jnp.clip(ba_ref[0, 0, :] + mu + lin_a, -10.0, 10.0)
    oh_ref[0, 0, :] = jnp.exp(log_h)
    oa_ref[0, 0, :] = jnp.exp(log_a)


def _tc_combine(XT_home, XT_away, beta_home, beta_away, mu, base_h, base_a):
    nblk = 8
    blk = BATCH // nblk
    out_h, out_a = pl.pallas_call(
        _tc_combine_kernel,
        grid=(nblk,),
        in_specs=[
            pl.BlockSpec((64, blk), lambda i: (0, i)),
            pl.BlockSpec((64, blk), lambda i: (0, i)),
            pl.BlockSpec((64, 1), lambda i: (0, 0)),
            pl.BlockSpec((64, 1), lambda i: (0, 0)),
            pl.BlockSpec((1, 1), lambda i: (0, 0)),
            pl.BlockSpec((1, 1, blk), lambda i: (i, 0, 0)),
            pl.BlockSpec((1, 1, blk), lambda i: (i, 0, 0)),
        ],
        out_specs=[
            pl.BlockSpec((1, 1, blk), lambda i: (i, 0, 0)),
            pl.BlockSpec((1, 1, blk), lambda i: (i, 0, 0)),
        ],
        out_shape=[
            jax.ShapeDtypeStruct((nblk, 1, blk), jnp.float32),
            jax.ShapeDtypeStruct((nblk, 1, blk), jnp.float32),
        ],
    )(XT_home, XT_away, beta_home.reshape(64, 1), beta_away.reshape(64, 1),
      mu.reshape(1, 1), base_h.reshape(nblk, 1, blk), base_a.reshape(nblk, 1, blk))
    return out_h.reshape(BATCH), out_a.reshape(BATCH)


def kernel(league_idx, season_idx, home_ts_idx, away_ts_idx, X_home, X_away,
           mu, gamma_league, hfa_league, delta_season, att, defn,
           beta_home, beta_away):
    lg = league_idx.astype(jnp.int32)
    sn = season_idx.astype(jnp.int32)
    hts = home_ts_idx.astype(jnp.int32)
    ats = away_ts_idx.astype(jnp.int32)

    base_h, base_a = _sc_gather(lg, sn, hts, ats, gamma_league, hfa_league,
                                delta_season, att, defn)
    mu_arr = jnp.asarray(mu, jnp.float32)
    return _tc_combine(X_home.T, X_away.T, beta_home, beta_away, mu_arr,
                       base_h, base_a)


# speedup vs baseline: 1.2071x; 1.2071x over previous
import jax
import jax.numpy as jnp
from jax import lax
from jax.experimental import pallas as pl
from jax.experimental.pallas import tpu as pltpu
from jax.experimental.pallas import tpu_sc as plsc

BATCH = 16384
LANES = 16

_NC = 2
_NS = 16
_NW = _NC * _NS
_CHUNK = BATCH // _NW


def _sc_gather_kernel(lg_hbm, sn_hbm, hts_hbm, ats_hbm,
                      gam_hbm, hfa_hbm, del_hbm, att_hbm, defn_hbm,
                      bh_hbm, ba_hbm,
                      lg_v, sn_v, hts_v, ats_v,
                      ah_v, aa_v, dh_v, da_v,
                      gam_t, hfa_t, del_t,
                      bh_v, ba_v, sem, sem2):
    wid = lax.axis_index("s") * _NC + lax.axis_index("c")
    base = wid * _CHUNK

    idx_copies = [
        pltpu.async_copy(lg_hbm.at[pl.ds(base, _CHUNK)], lg_v, sem2),
        pltpu.async_copy(sn_hbm.at[pl.ds(base, _CHUNK)], sn_v, sem2),
        pltpu.async_copy(hts_hbm.at[pl.ds(base, _CHUNK)], hts_v, sem2),
        pltpu.async_copy(ats_hbm.at[pl.ds(base, _CHUNK)], ats_v, sem2),
    ]
    tbl_copies = [
        pltpu.async_copy(gam_hbm, gam_t.at[pl.ds(0, 1000)], sem2),
        pltpu.async_copy(hfa_hbm, hfa_t.at[pl.ds(0, 1000)], sem2),
        pltpu.async_copy(del_hbm, del_t.at[pl.ds(0, 50)], sem2),
    ]
    for c in idx_copies:
        c.wait()

    copies = [
        pltpu.async_copy(att_hbm.at[hts_v], ah_v, sem),
        pltpu.async_copy(att_hbm.at[ats_v], aa_v, sem),
        pltpu.async_copy(defn_hbm.at[hts_v], dh_v, sem),
        pltpu.async_copy(defn_hbm.at[ats_v], da_v, sem),
    ]

    for c in tbl_copies:
        c.wait()
    for c in copies:
        c.wait()

    for i in range(_CHUNK // LANES):
        s = i * LANES
        lv = lg_v[pl.ds(s, LANES)]
        sv = sn_v[pl.ds(s, LANES)]
        g = plsc.load_gather(gam_t, [lv])
        h = plsc.load_gather(hfa_t, [lv])
        d = plsc.load_gather(del_t, [sv])
        gd = g + d
        bh_v[pl.ds(s, LANES)] = gd + h + ah_v[pl.ds(s, LANES)] - da_v[pl.ds(s, LANES)]
        ba_v[pl.ds(s, LANES)] = gd + aa_v[pl.ds(s, LANES)] - dh_v[pl.ds(s, LANES)]

    pltpu.sync_copy(bh_v, bh_hbm.at[pl.ds(base, _CHUNK)])
    pltpu.sync_copy(ba_v, ba_hbm.at[pl.ds(base, _CHUNK)])


def _sc_gather(lg, sn, hts, ats, gam2, hfa2, del2, att, defn):
    mesh = plsc.VectorSubcoreMesh(core_axis_name="c", subcore_axis_name="s")
    f = pl.kernel(
        _sc_gather_kernel,
        mesh=mesh,
        compiler_params=pltpu.CompilerParams(needs_layout_passes=False),
        out_type=[
            jax.ShapeDtypeStruct((BATCH,), jnp.float32),
            jax.ShapeDtypeStruct((BATCH,), jnp.float32),
        ],
        scratch_types=[
            pltpu.VMEM((_CHUNK,), jnp.int32),
            pltpu.VMEM((_CHUNK,), jnp.int32),
            pltpu.VMEM((_CHUNK,), jnp.int32),
            pltpu.VMEM((_CHUNK,), jnp.int32),
            pltpu.VMEM((_CHUNK,), jnp.float32),
            pltpu.VMEM((_CHUNK,), jnp.float32),
            pltpu.VMEM((_CHUNK,), jnp.float32),
            pltpu.VMEM((_CHUNK,), jnp.float32),
            pltpu.VMEM((1024,), jnp.float32),
            pltpu.VMEM((1024,), jnp.float32),
            pltpu.VMEM((64,), jnp.float32),
            pltpu.VMEM((_CHUNK,), jnp.float32),
            pltpu.VMEM((_CHUNK,), jnp.float32),
            pltpu.SemaphoreType.DMA,
            pltpu.SemaphoreType.DMA,
        ],
    )
    return f(lg, sn, hts, ats, gam2, hfa2, del2, att, defn)


def _tc_lin_kernel(xh_ref, xa_ref, beh_ref, bea_ref, lh_ref, la_ref):
    lh_ref[0, 0, :] = jnp.sum(xh_ref[...] * beh_ref[...], axis=0)
    la_ref[0, 0, :] = jnp.sum(xa_ref[...] * bea_ref[...], axis=0)


def _tc_lin(XT_home, XT_away, beta_home, beta_away):
    nblk = 8
    blk = BATCH // nblk
    lh, la = pl.pallas_call(
        _tc_lin_kernel,
        grid=(nblk,),
        in_specs=[
            pl.BlockSpec((64, blk), lambda i: (0, i)),
            pl.BlockSpec((64, blk), lambda i: (0, i)),
            pl.BlockSpec((64, 1), lambda i: (0, 0)),
            pl.BlockSpec((64, 1), lambda i: (0, 0)),
        ],
        out_specs=[
            pl.BlockSpec((1, 1, blk), lambda i: (i, 0, 0)),
            pl.BlockSpec((1, 1, blk), lambda i: (i, 0, 0)),
        ],
        out_shape=[
            jax.ShapeDtypeStruct((nblk, 1, blk), jnp.float32),
            jax.ShapeDtypeStruct((nblk, 1, blk), jnp.float32),
        ],
    )(XT_home, XT_away, beta_home.reshape(64, 1), beta_away.reshape(64, 1))
    return lh, la


def _tc_final_kernel(mu_ref, bh_ref, ba_ref, lh_ref, la_ref, oh_ref, oa_ref):
    mu = mu_ref[0, 0]
    log_h = jnp.clip(bh_ref[...] + mu + lh_ref[...], -10.0, 10.0)
    log_a = jnp.clip(ba_ref[...] + mu + la_ref[...], -10.0, 10.0)
    oh_ref[...] = jnp.exp(log_h)
    oa_ref[...] = jnp.exp(log_a)


def _tc_final(mu, base_h, base_a, lh, la):
    nblk = 8
    blk = BATCH // nblk
    bh3 = base_h.reshape(nblk, 1, blk)
    ba3 = base_a.reshape(nblk, 1, blk)
    out_h, out_a = pl.pallas_call(
        _tc_final_kernel,
        in_specs=[
            pl.BlockSpec((1, 1), lambda: (0, 0)),
            pl.BlockSpec((nblk, 1, blk), lambda: (0, 0, 0)),
            pl.BlockSpec((nblk, 1, blk), lambda: (0, 0, 0)),
            pl.BlockSpec((nblk, 1, blk), lambda: (0, 0, 0)),
            pl.BlockSpec((nblk, 1, blk), lambda: (0, 0, 0)),
        ],
        out_specs=[
            pl.BlockSpec((nblk, 1, blk), lambda: (0, 0, 0)),
            pl.BlockSpec((nblk, 1, blk), lambda: (0, 0, 0)),
        ],
        out_shape=[
            jax.ShapeDtypeStruct((nblk, 1, blk), jnp.float32),
            jax.ShapeDtypeStruct((nblk, 1, blk), jnp.float32),
        ],
    )(mu.reshape(1, 1), bh3, ba3, lh, la)
    return out_h.reshape(BATCH), out_a.reshape(BATCH)


def kernel(league_idx, season_idx, home_ts_idx, away_ts_idx, X_home, X_away,
           mu, gamma_league, hfa_league, delta_season, att, defn,
           beta_home, beta_away):
    lg = league_idx.astype(jnp.int32)
    sn = season_idx.astype(jnp.int32)
    hts = home_ts_idx.astype(jnp.int32)
    ats = away_ts_idx.astype(jnp.int32)

    base_h, base_a = _sc_gather(lg, sn, hts, ats, gamma_league, hfa_league,
                                delta_season, att, defn)
    lh, la = _tc_lin(X_home.T, X_away.T, beta_home, beta_away)

    mu_arr = jnp.asarray(mu, jnp.float32)
    return _tc_final(mu_arr, base_h, base_a, lh, la)
